# baseline (speedup 1.0000x reference)
import functools

import jax
import jax.numpy as jnp
from jax import lax
from jax.experimental import pallas as pl
from jax.experimental.pallas import tpu as pltpu
from jax.experimental.pallas import tpu_sc as plsc

N = 10000
E = 320000
F_IN = 128
H = 128
G = 64

NC = 2
NS = 16
NW = NC * NS
EPW = E // NW
CHUNK = 40
NCHUNK = EPW // CHUNK
NB = 8
NI = 8
UNROLL = 8
GDEPTH = 3
IDEPTH = GDEPTH + 1
ZR = 40
NZC = N // ZR
ZPT = -(-NZC // NS)

BR = 2000


def _sc_agg_body(h_hbm, src_hbm, dst_hbm, out_hbm,
                 acc, sidx, didx, rows, stage, isem, gsem, ssem):
    c = lax.axis_index("c")
    s = lax.axis_index("s")
    wid = s * NC + c

    def _zrow(i, carry):
        def _zcol(j, carry2):
            stage[i, pl.ds(j * 16, 16)] = jnp.zeros((16,), jnp.float32)
            return carry2
        return lax.fori_loop(0, H // 16, _zcol, carry)
    lax.fori_loop(0, ZR, _zrow, 0)

    def _zacc(j, carry):
        k = s + j * NS
        @pl.when(k < NZC)
        def _():
            pltpu.sync_copy(stage, acc.at[pl.ds(k * ZR, ZR)])
        return carry
    lax.fori_loop(0, ZPT, _zacc, 0)
    plsc.subcore_barrier()

    ebase = wid * EPW

    def _start_idx(i, si):
        pltpu.async_copy(src_hbm.at[pl.ds(ebase + i * CHUNK, CHUNK)],
                         sidx.at[si], isem.at[si])
        pltpu.async_copy(dst_hbm.at[pl.ds(ebase + i * CHUNK, CHUNK)],
                         didx.at[si], isem.at[si])

    def _wait_idx(si):
        pltpu.make_async_copy(src_hbm.at[pl.ds(ebase, CHUNK)],
                              sidx.at[si], isem.at[si]).wait()
        pltpu.make_async_copy(dst_hbm.at[pl.ds(ebase, CHUNK)],
                              didx.at[si], isem.at[si]).wait()

    def _start_g(si, sr):
        pltpu.async_copy(h_hbm.at[sidx.at[si]], rows.at[sr], gsem.at[sr])

    def _wait_g(si, sr):
        pltpu.make_async_copy(h_hbm.at[sidx.at[si]], rows.at[sr],
                              gsem.at[sr]).wait()

    def _start_s(si, sr):
        pltpu.async_copy(rows.at[sr], acc.at[didx.at[si]], ssem.at[sr],
                         add=True)

    def _wait_s(si, sr):
        pltpu.make_async_copy(rows.at[sr], acc.at[didx.at[si]],
                              ssem.at[sr]).wait()

    for i in range(IDEPTH):
        _start_idx(i, i % NI)
    for i in range(GDEPTH):
        _wait_idx(i % NI)
        _start_g(i % NI, i % NB)

    NTAIL = NCHUNK % UNROLL + UNROLL * ((NCHUNK % UNROLL) < GDEPTH)
    NOUT = (NCHUNK - NTAIL) // UNROLL

    def _chunk_body(i, u, io):
        _wait_g(u % NI, u % NB)
        _start_s(u % NI, u % NB)
        _start_idx(i + IDEPTH, (u + IDEPTH) % NI)
        if u >= GDEPTH:
            _wait_s((u - GDEPTH) % NI, (u - GDEPTH) % NB)
        else:
            @pl.when(io > 0)
            def _():
                _wait_s((u - GDEPTH) % NI, (u - GDEPTH) % NB)
        _wait_idx((u + GDEPTH) % NI)
        _start_g((u + GDEPTH) % NI, (u + GDEPTH) % NB)

    def _outer(io, carry):
        for u in range(UNROLL):
            _chunk_body(io * UNROLL + u, u, io)
        return carry
    lax.fori_loop(0, NOUT, _outer, 0)

    for i in range(NCHUNK - NTAIL, NCHUNK):
        _wait_g(i % NI, i % NB)
        _start_s(i % NI, i % NB)
        if i + IDEPTH < NCHUNK:
            _start_idx(i + IDEPTH, (i + IDEPTH) % NI)
        if i - GDEPTH >= 0:
            _wait_s((i - GDEPTH) % NI, (i - GDEPTH) % NB)
        if i + GDEPTH < NCHUNK:
            _wait_idx((i + GDEPTH) % NI)
            _start_g((i + GDEPTH) % NI, (i + GDEPTH) % NB)
    for i in range(NCHUNK - GDEPTH, NCHUNK):
        _wait_s(i % NI, i % NB)
    plsc.subcore_barrier()

    def _wout(j, carry):
        k = s + j * NS
        @pl.when(k < NZC)
        def _():
            pltpu.sync_copy(acc.at[pl.ds(k * ZR, ZR)], stage)
            pltpu.sync_copy(stage, out_hbm.at[c, pl.ds(k * ZR, ZR)])
        return carry
    lax.fori_loop(0, ZPT, _wout, 0)


@functools.cache
def _build_sc_agg():
    return pl.kernel(
        _sc_agg_body,
        out_type=jax.ShapeDtypeStruct((NC, N, H), jnp.float32),
        mesh=plsc.VectorSubcoreMesh(core_axis_name="c", subcore_axis_name="s"),
        scratch_types=[
            pltpu.VMEM_SHARED((N, H), jnp.float32),
            pltpu.VMEM((NI, CHUNK), jnp.int32),
            pltpu.VMEM((NI, CHUNK), jnp.int32),
            pltpu.VMEM((NB, CHUNK, H), jnp.float32),
            pltpu.VMEM((ZR, H), jnp.float32),
            pltpu.SemaphoreType.DMA((NI,)),
            pltpu.SemaphoreType.DMA((NB,)),
            pltpu.SemaphoreType.DMA((NB,)),
        ],
    )


def _sc_agg(h, src3, dst3):
    return _build_sc_agg()(h, src3, dst3)


def _lin_relu_body(x_ref, w_ref, b_ref, o_ref):
    o_ref[...] = jnp.maximum(
        jnp.dot(x_ref[...], w_ref[...], preferred_element_type=jnp.float32)
        + b_ref[...], 0.0)


def _lin_relu(x, w, b):
    return pl.pallas_call(
        _lin_relu_body,
        grid=(N // BR,),
        in_specs=[
            pl.BlockSpec((BR, F_IN), lambda i: (i, 0)),
            pl.BlockSpec((F_IN, H), lambda i: (0, 0)),
            pl.BlockSpec((1, H), lambda i: (0, 0)),
        ],
        out_specs=pl.BlockSpec((BR, H), lambda i: (i, 0)),
        out_shape=jax.ShapeDtypeStruct((N, H), jnp.float32),
    )(x, w, b.reshape(1, H))


def _gin_body(h_ref, a0_ref, a1_ref, w1_ref, b1_ref, g_ref, bb_ref, m_ref,
              v_ref, w2_ref, b2_ref, o_ref):
    z = h_ref[...] + a0_ref[...] + a1_ref[...]
    t = jnp.dot(z, w1_ref[...], preferred_element_type=jnp.float32) + b1_ref[...]
    sc = g_ref[...] * lax.rsqrt(v_ref[...] + 1e-5)
    t = jnp.maximum(t * sc + (bb_ref[...] - m_ref[...] * sc), 0.0)
    o_ref[...] = jnp.maximum(
        jnp.dot(t, w2_ref[...], preferred_element_type=jnp.float32)
        + b2_ref[...], 0.0)


def _gin(h, a0, a1, w1, b1, bg, bb, bm, bv, w2, b2):
    h2 = 2 * H
    return pl.pallas_call(
        _gin_body,
        grid=(N // BR,),
        in_specs=[
            pl.BlockSpec((BR, H), lambda i: (i, 0)),
            pl.BlockSpec((BR, H), lambda i: (i, 0)),
            pl.BlockSpec((BR, H), lambda i: (i, 0)),
            pl.BlockSpec((H, h2), lambda i: (0, 0)),
            pl.BlockSpec((1, h2), lambda i: (0, 0)),
            pl.BlockSpec((1, h2), lambda i: (0, 0)),
            pl.BlockSpec((1, h2), lambda i: (0, 0)),
            pl.BlockSpec((1, h2), lambda i: (0, 0)),
            pl.BlockSpec((1, h2), lambda i: (0, 0)),
            pl.BlockSpec((h2, H), lambda i: (0, 0)),
            pl.BlockSpec((1, H), lambda i: (0, 0)),
        ],
        out_specs=pl.BlockSpec((BR, H), lambda i: (i, 0)),
        out_shape=jax.ShapeDtypeStruct((N, H), jnp.float32),
    )(h, a0, a1, w1, b1.reshape(1, h2), bg.reshape(1, h2), bb.reshape(1, h2),
      bm.reshape(1, h2), bv.reshape(1, h2), w2, b2.reshape(1, H))


def _pool_body(x_ref, x1_ref, x2_ref, wa_ref, wb_ref, wc_ref, sb_ref,
               batch_ref, attw_ref, c1w_ref, c1b_ref, cg_ref, cb_ref,
               cm_ref, cv_ref, c2w_ref, c2b_ref, c3w_ref, c3b_ref, o_ref):
    xs = (
        jnp.dot(x_ref[...], wa_ref[...], preferred_element_type=jnp.float32)
        + jnp.dot(x1_ref[...], wb_ref[...], preferred_element_type=jnp.float32)
        + jnp.dot(x2_ref[...], wc_ref[...], preferred_element_type=jnp.float32)
        + sb_ref[...])
    scores = lax.dot_general(attw_ref[...], xs, (((1,), (1,)), ((), ())),
                             preferred_element_type=jnp.float32)
    gid = lax.broadcasted_iota(jnp.int32, (G, N), 0)
    m = batch_ref[...] == gid
    sm = jnp.where(m, scores, -1e30)
    smax = jnp.max(sm, axis=1, keepdims=True)
    e = jnp.where(m, jnp.exp(sm - smax), 0.0)
    den = jnp.sum(e, axis=1, keepdims=True)
    aw = e / (den + 1e-16)
    x_att = jnp.dot(aw, xs, preferred_element_type=jnp.float32)
    mf = m.astype(jnp.float32)
    cnt = jnp.sum(mf, axis=1, keepdims=True)
    x_mean = (jnp.dot(mf, xs, preferred_element_type=jnp.float32)
              / jnp.maximum(cnt, 1.0))
    z = jnp.concatenate([x_att, x_mean], axis=1)
    sc1 = cg_ref[...] * lax.rsqrt(cv_ref[...] + 1e-5)
    z = jnp.maximum(
        (jnp.dot(z, c1w_ref[...], preferred_element_type=jnp.float32)
         + c1b_ref[...]) * sc1 + (cb_ref[...] - cm_ref[...] * sc1), 0.0)
    z = jnp.maximum(
        jnp.dot(z, c2w_ref[...], preferred_element_type=jnp.float32)
        + c2b_ref[...], 0.0)
    o_ref[...] = (jnp.dot(z, c3w_ref[...], preferred_element_type=jnp.float32)
                  + c3b_ref[...])


def _pool(x, x1, x2, wa, wb, wc, sb, batch2d, att_w, c1w, c1b, cg, cb, cm,
          cv, c2w, c2b, c3w, c3b):
    h2 = 2 * H
    full = lambda shape: pl.BlockSpec(shape, lambda: (0,) * len(shape))
    return pl.pallas_call(
        _pool_body,
        in_specs=[
            full((N, F_IN)), full((N, H)), full((N, H)),
            full((F_IN, H)), full((H, H)), full((H, H)), full((1, H)),
            full((1, N)), full((1, H)),
            full((h2, H)), full((1, H)),
            full((1, H)), full((1, H)), full((1, H)), full((1, H)),
            full((H, H // 2)), full((1, H // 2)),
            full((H // 2, 2)), full((1, 2)),
        ],
        out_specs=full((G, 2)),
        out_shape=jax.ShapeDtypeStruct((G, 2), jnp.float32),
    )(x, x1, x2, wa, wb, wc, sb.reshape(1, H), batch2d, att_w, c1w,
      c1b.reshape(1, H), cg.reshape(1, H), cb.reshape(1, H),
      cm.reshape(1, H), cv.reshape(1, H), c2w,
      c2b.reshape(1, H // 2), c3w, c3b.reshape(1, 2))


def kernel(x, edge_index, batch, inp_W, inp_b, g1_l1_W, g1_l1_b, g1_bn_g,
           g1_bn_b, g1_bn_m, g1_bn_v, g1_l2_W, g1_l2_b, g2_l1_W, g2_l1_b,
           g2_bn_g, g2_bn_b, g2_bn_m, g2_bn_v, g2_l2_W, g2_l2_b, skip_W,
           skip_b, att_w, c1_W, c1_b, cbn_g, cbn_b, cbn_m, cbn_v, c2_W, c2_b,
           c3_W, c3_b):
    src_flat = edge_index[0]
    dst_flat = edge_index[1]

    h = _lin_relu(x, inp_W, inp_b)
    p1 = _sc_agg(h, src_flat, dst_flat)
    x1 = _gin(h, p1[0], p1[1], g1_l1_W, g1_l1_b, g1_bn_g, g1_bn_b, g1_bn_m,
              g1_bn_v, g1_l2_W, g1_l2_b)
    p2 = _sc_agg(x1, src_flat, dst_flat)
    x2 = _gin(x1, p2[0], p2[1], g2_l1_W, g2_l1_b, g2_bn_g, g2_bn_b, g2_bn_m,
              g2_bn_v, g2_l2_W, g2_l2_b)
    return _pool(x, x1, x2, skip_W[:F_IN], skip_W[F_IN:F_IN + H],
                 skip_W[F_IN + H:], skip_b, batch.reshape(1, N), att_w, c1_W,
                 c1_b, cbn_g, cbn_b, cbn_m, cbn_v, c2_W, c2_b, c3_W, c3_b)


# speedup vs baseline: 1.0895x; 1.0895x over previous
import functools

import jax
import jax.numpy as jnp
from jax import lax
from jax.experimental import pallas as pl
from jax.experimental.pallas import tpu as pltpu
from jax.experimental.pallas import tpu_sc as plsc

N = 10000
E = 320000
F_IN = 128
H = 128
G = 64

NC = 2
NS = 16
NW = NC * NS
EPW = E // NW
CHUNK = 80
NCHUNK = EPW // CHUNK
NB = 4
NI = 8
UNROLL = NI
ZR = 40
NZC = N // ZR
ZPT = -(-NZC // NS)

BR = 2000


def _sc_agg_body(h_hbm, src_hbm, dst_hbm, out_hbm,
                 acc, sidx, didx, rows, stage, isem, gsem, ssem):
    c = lax.axis_index("c")
    s = lax.axis_index("s")
    wid = s * NC + c

    def _zrow(i, carry):
        def _zcol(j, carry2):
            stage[i, pl.ds(j * 16, 16)] = jnp.zeros((16,), jnp.float32)
            return carry2
        return lax.fori_loop(0, H // 16, _zcol, carry)
    lax.fori_loop(0, ZR, _zrow, 0)

    def _zacc(j, carry):
        k = s + j * NS
        @pl.when(k < NZC)
        def _():
            pltpu.sync_copy(stage, acc.at[pl.ds(k * ZR, ZR)])
        return carry
    lax.fori_loop(0, ZPT, _zacc, 0)
    plsc.subcore_barrier()

    ebase = wid * EPW

    def _start_idx(i, si):
        pltpu.async_copy(src_hbm.at[pl.ds(ebase + i * CHUNK, CHUNK)],
                         sidx.at[si], isem.at[si])
        pltpu.async_copy(dst_hbm.at[pl.ds(ebase + i * CHUNK, CHUNK)],
                         didx.at[si], isem.at[si])

    def _wait_idx(si):
        pltpu.make_async_copy(src_hbm.at[pl.ds(ebase, CHUNK)],
                              sidx.at[si], isem.at[si]).wait()
        pltpu.make_async_copy(dst_hbm.at[pl.ds(ebase, CHUNK)],
                              didx.at[si], isem.at[si]).wait()

    def _start_g(si, sr):
        pltpu.async_copy(h_hbm.at[sidx.at[si]], rows.at[sr], gsem.at[sr])

    def _wait_g(si, sr):
        pltpu.make_async_copy(h_hbm.at[sidx.at[si]], rows.at[sr],
                              gsem.at[sr]).wait()

    def _start_s(si, sr):
        pltpu.async_copy(rows.at[sr], acc.at[didx.at[si]], ssem.at[sr],
                         add=True)

    def _wait_s(si, sr):
        pltpu.make_async_copy(rows.at[sr], acc.at[didx.at[si]],
                              ssem.at[sr]).wait()

    _start_idx(0, 0)
    _start_idx(1, 1)
    _start_idx(2, 2)
    _wait_idx(0)
    _start_g(0, 0)
    _wait_idx(1)
    _start_g(1, 1)

    NTAIL = 5
    NOUT = (NCHUNK - NTAIL) // UNROLL

    def _chunk_body(i, u, io):
        _wait_g(u, u % NB)
        _start_s(u, u % NB)
        _start_idx(i + 3, (u + 3) % NI)
        if u >= 2:
            _wait_s((u - 2) % NI, (u - 2) % NB)
        else:
            @pl.when(io > 0)
            def _():
                _wait_s((u - 2) % NI, (u - 2) % NB)
        _wait_idx((u + 2) % NI)
        _start_g((u + 2) % NI, (u + 2) % NB)

    def _outer(io, carry):
        for u in range(UNROLL):
            _chunk_body(io * UNROLL + u, u, io)
        return carry
    lax.fori_loop(0, NOUT, _outer, 0)

    for i in range(NCHUNK - NTAIL, NCHUNK):
        _wait_g(i % NI, i % NB)
        _start_s(i % NI, i % NB)
        if i + 3 < NCHUNK:
            _start_idx(i + 3, (i + 3) % NI)
        _wait_s((i - 2) % NI, (i - 2) % NB)
        if i + 2 < NCHUNK:
            _wait_idx((i + 2) % NI)
            _start_g((i + 2) % NI, (i + 2) % NB)
    _wait_s((NCHUNK - 2) % NI, (NCHUNK - 2) % NB)
    _wait_s((NCHUNK - 1) % NI, (NCHUNK - 1) % NB)
    plsc.subcore_barrier()

    def _wout(j, carry):
        k = s + j * NS
        @pl.when(k < NZC)
        def _():
            pltpu.sync_copy(acc.at[pl.ds(k * ZR, ZR)], stage)
            pltpu.sync_copy(stage, out_hbm.at[c, pl.ds(k * ZR, ZR)])
        return carry
    lax.fori_loop(0, ZPT, _wout, 0)


@functools.cache
def _build_sc_agg():
    return pl.kernel(
        _sc_agg_body,
        out_type=jax.ShapeDtypeStruct((NC, N, H), jnp.float32),
        mesh=plsc.VectorSubcoreMesh(core_axis_name="c", subcore_axis_name="s"),
        scratch_types=[
            pltpu.VMEM_SHARED((N, H), jnp.float32),
            pltpu.VMEM((NI, CHUNK), jnp.int32),
            pltpu.VMEM((NI, CHUNK), jnp.int32),
            pltpu.VMEM((NB, CHUNK, H), jnp.float32),
            pltpu.VMEM((ZR, H), jnp.float32),
            pltpu.SemaphoreType.DMA((NI,)),
            pltpu.SemaphoreType.DMA((NB,)),
            pltpu.SemaphoreType.DMA((NB,)),
        ],
    )


def _sc_agg(h, src3, dst3):
    return _build_sc_agg()(h, src3, dst3)


def _lin_relu_body(x_ref, w_ref, b_ref, o_ref):
    o_ref[...] = jnp.maximum(
        jnp.dot(x_ref[...], w_ref[...], preferred_element_type=jnp.float32)
        + b_ref[...], 0.0)


def _lin_relu(x, w, b):
    return pl.pallas_call(
        _lin_relu_body,
        grid=(N // BR,),
        in_specs=[
            pl.BlockSpec((BR, F_IN), lambda i: (i, 0)),
            pl.BlockSpec((F_IN, H), lambda i: (0, 0)),
            pl.BlockSpec((1, H), lambda i: (0, 0)),
        ],
        out_specs=pl.BlockSpec((BR, H), lambda i: (i, 0)),
        out_shape=jax.ShapeDtypeStruct((N, H), jnp.float32),
    )(x, w, b.reshape(1, H))


def _gin_body(h_ref, a0_ref, a1_ref, w1_ref, b1_ref, g_ref, bb_ref, m_ref,
              v_ref, w2_ref, b2_ref, o_ref):
    z = h_ref[...] + a0_ref[...] + a1_ref[...]
    t = jnp.dot(z, w1_ref[...], preferred_element_type=jnp.float32) + b1_ref[...]
    sc = g_ref[...] * lax.rsqrt(v_ref[...] + 1e-5)
    t = jnp.maximum(t * sc + (bb_ref[...] - m_ref[...] * sc), 0.0)
    o_ref[...] = jnp.maximum(
        jnp.dot(t, w2_ref[...], preferred_element_type=jnp.float32)
        + b2_ref[...], 0.0)


def _gin(h, a0, a1, w1, b1, bg, bb, bm, bv, w2, b2):
    h2 = 2 * H
    return pl.pallas_call(
        _gin_body,
        grid=(N // BR,),
        in_specs=[
            pl.BlockSpec((BR, H), lambda i: (i, 0)),
            pl.BlockSpec((BR, H), lambda i: (i, 0)),
            pl.BlockSpec((BR, H), lambda i: (i, 0)),
            pl.BlockSpec((H, h2), lambda i: (0, 0)),
            pl.BlockSpec((1, h2), lambda i: (0, 0)),
            pl.BlockSpec((1, h2), lambda i: (0, 0)),
            pl.BlockSpec((1, h2), lambda i: (0, 0)),
            pl.BlockSpec((1, h2), lambda i: (0, 0)),
            pl.BlockSpec((1, h2), lambda i: (0, 0)),
            pl.BlockSpec((h2, H), lambda i: (0, 0)),
            pl.BlockSpec((1, H), lambda i: (0, 0)),
        ],
        out_specs=pl.BlockSpec((BR, H), lambda i: (i, 0)),
        out_shape=jax.ShapeDtypeStruct((N, H), jnp.float32),
    )(h, a0, a1, w1, b1.reshape(1, h2), bg.reshape(1, h2), bb.reshape(1, h2),
      bm.reshape(1, h2), bv.reshape(1, h2), w2, b2.reshape(1, H))


def _pool_body(x_ref, x1_ref, x2_ref, wa_ref, wb_ref, wc_ref, sb_ref,
               batch_ref, attw_ref, c1w_ref, c1b_ref, cg_ref, cb_ref,
               cm_ref, cv_ref, c2w_ref, c2b_ref, c3w_ref, c3b_ref, o_ref):
    xs = (
        jnp.dot(x_ref[...], wa_ref[...], preferred_element_type=jnp.float32)
        + jnp.dot(x1_ref[...], wb_ref[...], preferred_element_type=jnp.float32)
        + jnp.dot(x2_ref[...], wc_ref[...], preferred_element_type=jnp.float32)
        + sb_ref[...])
    scores = lax.dot_general(attw_ref[...], xs, (((1,), (1,)), ((), ())),
                             preferred_element_type=jnp.float32)
    gid = lax.broadcasted_iota(jnp.int32, (G, N), 0)
    m = batch_ref[...] == gid
    sm = jnp.where(m, scores, -1e30)
    smax = jnp.max(sm, axis=1, keepdims=True)
    e = jnp.where(m, jnp.exp(sm - smax), 0.0)
    den = jnp.sum(e, axis=1, keepdims=True)
    aw = e / (den + 1e-16)
    x_att = jnp.dot(aw, xs, preferred_element_type=jnp.float32)
    mf = m.astype(jnp.float32)
    cnt = jnp.sum(mf, axis=1, keepdims=True)
    x_mean = (jnp.dot(mf, xs, preferred_element_type=jnp.float32)
              / jnp.maximum(cnt, 1.0))
    z = jnp.concatenate([x_att, x_mean], axis=1)
    sc1 = cg_ref[...] * lax.rsqrt(cv_ref[...] + 1e-5)
    z = jnp.maximum(
        (jnp.dot(z, c1w_ref[...], preferred_element_type=jnp.float32)
         + c1b_ref[...]) * sc1 + (cb_ref[...] - cm_ref[...] * sc1), 0.0)
    z = jnp.maximum(
        jnp.dot(z, c2w_ref[...], preferred_element_type=jnp.float32)
        + c2b_ref[...], 0.0)
    o_ref[...] = (jnp.dot(z, c3w_ref[...], preferred_element_type=jnp.float32)
                  + c3b_ref[...])


def _pool(x, x1, x2, wa, wb, wc, sb, batch2d, att_w, c1w, c1b, cg, cb, cm,
          cv, c2w, c2b, c3w, c3b):
    h2 = 2 * H
    full = lambda shape: pl.BlockSpec(shape, lambda: (0,) * len(shape))
    return pl.pallas_call(
        _pool_body,
        in_specs=[
            full((N, F_IN)), full((N, H)), full((N, H)),
            full((F_IN, H)), full((H, H)), full((H, H)), full((1, H)),
            full((1, N)), full((1, H)),
            full((h2, H)), full((1, H)),
            full((1, H)), full((1, H)), full((1, H)), full((1, H)),
            full((H, H // 2)), full((1, H // 2)),
            full((H // 2, 2)), full((1, 2)),
        ],
        out_specs=full((G, 2)),
        out_shape=jax.ShapeDtypeStruct((G, 2), jnp.float32),
    )(x, x1, x2, wa, wb, wc, sb.reshape(1, H), batch2d, att_w, c1w,
      c1b.reshape(1, H), cg.reshape(1, H), cb.reshape(1, H),
      cm.reshape(1, H), cv.reshape(1, H), c2w,
      c2b.reshape(1, H // 2), c3w, c3b.reshape(1, 2))


def kernel(x, edge_index, batch, inp_W, inp_b, g1_l1_W, g1_l1_b, g1_bn_g,
           g1_bn_b, g1_bn_m, g1_bn_v, g1_l2_W, g1_l2_b, g2_l1_W, g2_l1_b,
           g2_bn_g, g2_bn_b, g2_bn_m, g2_bn_v, g2_l2_W, g2_l2_b, skip_W,
           skip_b, att_w, c1_W, c1_b, cbn_g, cbn_b, cbn_m, cbn_v, c2_W, c2_b,
           c3_W, c3_b):
    src_flat = edge_index[0]
    dst_flat = edge_index[1]

    h = _lin_relu(x, inp_W, inp_b)
    p1 = _sc_agg(h, src_flat, dst_flat)
    x1 = _gin(h, p1[0], p1[1], g1_l1_W, g1_l1_b, g1_bn_g, g1_bn_b, g1_bn_m,
              g1_bn_v, g1_l2_W, g1_l2_b)
    p2 = _sc_agg(x1, src_flat, dst_flat)
    x2 = _gin(x1, p2[0], p2[1], g2_l1_W, g2_l1_b, g2_bn_g, g2_bn_b, g2_bn_m,
              g2_bn_v, g2_l2_W, g2_l2_b)
    return _pool(x, x1, x2, skip_W[:F_IN], skip_W[F_IN:F_IN + H],
                 skip_W[F_IN + H:], skip_b, batch.reshape(1, N), att_w, c1_W,
                 c1_b, cbn_g, cbn_b, cbn_m, cbn_v, c2_W, c2_b, c3_W, c3_b)


# speedup vs baseline: 1.1150x; 1.0234x over previous
import functools

import jax
import jax.numpy as jnp
from jax import lax
from jax.experimental import pallas as pl
from jax.experimental.pallas import tpu as pltpu
from jax.experimental.pallas import tpu_sc as plsc

N = 10000
E = 320000
F_IN = 128
H = 128
G = 64

NC = 2
NS = 16
NW = NC * NS
EPW = E // NW
CHUNK = 80
NCHUNK = EPW // CHUNK
NB = 4
NI = 8
UNROLL = NI
ZR = 40
NZC = N // ZR
ZPT = -(-NZC // NS)

BR = 2000


def _sc_agg_body(h_hbm, src_hbm, dst_hbm, out_hbm,
                 acc, sidx, didx, rows, stage, isem, gsem, ssem):
    c = lax.axis_index("c")
    s = lax.axis_index("s")
    wid = s * NC + c

    def _zrow(i, carry):
        def _zcol(j, carry2):
            stage[i, pl.ds(j * 16, 16)] = jnp.zeros((16,), jnp.float32)
            return carry2
        return lax.fori_loop(0, H // 16, _zcol, carry)
    lax.fori_loop(0, ZR, _zrow, 0)

    def _zacc(j, carry):
        k = s + j * NS
        @pl.when(k < NZC)
        def _():
            pltpu.sync_copy(stage, acc.at[pl.ds(k * ZR, ZR)])
        return carry
    lax.fori_loop(0, ZPT, _zacc, 0)
    plsc.subcore_barrier()

    ebase = wid * EPW

    def _start_idx(i, si):
        pltpu.async_copy(src_hbm.at[pl.ds(ebase + i * CHUNK, CHUNK)],
                         sidx.at[si], isem.at[si])
        pltpu.async_copy(dst_hbm.at[pl.ds(ebase + i * CHUNK, CHUNK)],
                         didx.at[si], isem.at[si])

    def _wait_idx(si):
        pltpu.make_async_copy(src_hbm.at[pl.ds(ebase, CHUNK)],
                              sidx.at[si], isem.at[si]).wait()
        pltpu.make_async_copy(dst_hbm.at[pl.ds(ebase, CHUNK)],
                              didx.at[si], isem.at[si]).wait()

    def _start_g(si, sr):
        pltpu.async_copy(h_hbm.at[sidx.at[si]], rows.at[sr], gsem.at[sr])

    def _wait_g(si, sr):
        pltpu.make_async_copy(h_hbm.at[sidx.at[si]], rows.at[sr],
                              gsem.at[sr]).wait()

    def _start_s(si, sr):
        pltpu.async_copy(rows.at[sr], acc.at[didx.at[si]], ssem.at[sr],
                         add=True)

    def _wait_s(si, sr):
        pltpu.make_async_copy(rows.at[sr], acc.at[didx.at[si]],
                              ssem.at[sr]).wait()

    _start_idx(0, 0)
    _start_idx(1, 1)
    _start_idx(2, 2)
    _wait_idx(0)
    _start_g(0, 0)
    _wait_idx(1)
    _start_g(1, 1)

    NTAIL = 5
    NOUT = (NCHUNK - NTAIL) // UNROLL

    def _chunk_body(i, u, io):
        _wait_g(u, u % NB)
        _start_s(u, u % NB)
        _start_idx(i + 3, (u + 3) % NI)
        if u >= 2:
            _wait_s((u - 2) % NI, (u - 2) % NB)
        else:
            @pl.when(io > 0)
            def _():
                _wait_s((u - 2) % NI, (u - 2) % NB)
        _wait_idx((u + 2) % NI)
        _start_g((u + 2) % NI, (u + 2) % NB)

    def _outer(io, carry):
        for u in range(UNROLL):
            _chunk_body(io * UNROLL + u, u, io)
        return carry
    lax.fori_loop(0, NOUT, _outer, 0)

    for i in range(NCHUNK - NTAIL, NCHUNK):
        _wait_g(i % NI, i % NB)
        _start_s(i % NI, i % NB)
        if i + 3 < NCHUNK:
            _start_idx(i + 3, (i + 3) % NI)
        _wait_s((i - 2) % NI, (i - 2) % NB)
        if i + 2 < NCHUNK:
            _wait_idx((i + 2) % NI)
            _start_g((i + 2) % NI, (i + 2) % NB)
    _wait_s((NCHUNK - 2) % NI, (NCHUNK - 2) % NB)
    _wait_s((NCHUNK - 1) % NI, (NCHUNK - 1) % NB)
    plsc.subcore_barrier()

    def _wout(j, carry):
        k = s + j * NS
        @pl.when(k < NZC)
        def _():
            pltpu.sync_copy(acc.at[pl.ds(k * ZR, ZR)], stage)
            pltpu.sync_copy(stage, out_hbm.at[c, pl.ds(k * ZR, ZR)])
        return carry
    lax.fori_loop(0, ZPT, _wout, 0)


@functools.cache
def _build_sc_agg():
    return pl.kernel(
        _sc_agg_body,
        out_type=jax.ShapeDtypeStruct((NC, N, H), jnp.float32),
        mesh=plsc.VectorSubcoreMesh(core_axis_name="c", subcore_axis_name="s"),
        scratch_types=[
            pltpu.VMEM_SHARED((N, H), jnp.float32),
            pltpu.VMEM((NI, CHUNK), jnp.int32),
            pltpu.VMEM((NI, CHUNK), jnp.int32),
            pltpu.VMEM((NB, CHUNK, H), jnp.float32),
            pltpu.VMEM((ZR, H), jnp.float32),
            pltpu.SemaphoreType.DMA((NI,)),
            pltpu.SemaphoreType.DMA((NB,)),
            pltpu.SemaphoreType.DMA((NB,)),
        ],
    )


def _sc_agg(h, src3, dst3):
    return _build_sc_agg()(h, src3, dst3)


def _lin_relu_body(x_ref, w_ref, b_ref, o_ref):
    o_ref[...] = jnp.maximum(
        jnp.dot(x_ref[...], w_ref[...], preferred_element_type=jnp.float32)
        + b_ref[...], 0.0)


def _lin_relu(x, w, b):
    return pl.pallas_call(
        _lin_relu_body,
        grid=(N // BR,),
        in_specs=[
            pl.BlockSpec((BR, F_IN), lambda i: (i, 0)),
            pl.BlockSpec((F_IN, H), lambda i: (0, 0)),
            pl.BlockSpec((1, H), lambda i: (0, 0)),
        ],
        out_specs=pl.BlockSpec((BR, H), lambda i: (i, 0)),
        out_shape=jax.ShapeDtypeStruct((N, H), jnp.float32),
    )(x, w, b.reshape(1, H))


def _gin_body(h_ref, a0_ref, a1_ref, w1_ref, b1_ref, g_ref, bb_ref, m_ref,
              v_ref, w2_ref, b2_ref, o_ref):
    z = h_ref[...] + a0_ref[...] + a1_ref[...]
    t = jnp.dot(z, w1_ref[...], preferred_element_type=jnp.float32) + b1_ref[...]
    sc = g_ref[...] * lax.rsqrt(v_ref[...] + 1e-5)
    t = jnp.maximum(t * sc + (bb_ref[...] - m_ref[...] * sc), 0.0)
    o_ref[...] = jnp.maximum(
        jnp.dot(t, w2_ref[...], preferred_element_type=jnp.float32)
        + b2_ref[...], 0.0)


def _gin(h, a0, a1, w1, b1, bg, bb, bm, bv, w2, b2):
    h2 = 2 * H
    return pl.pallas_call(
        _gin_body,
        grid=(N // BR,),
        in_specs=[
            pl.BlockSpec((BR, H), lambda i: (i, 0)),
            pl.BlockSpec((BR, H), lambda i: (i, 0)),
            pl.BlockSpec((BR, H), lambda i: (i, 0)),
            pl.BlockSpec((H, h2), lambda i: (0, 0)),
            pl.BlockSpec((1, h2), lambda i: (0, 0)),
            pl.BlockSpec((1, h2), lambda i: (0, 0)),
            pl.BlockSpec((1, h2), lambda i: (0, 0)),
            pl.BlockSpec((1, h2), lambda i: (0, 0)),
            pl.BlockSpec((1, h2), lambda i: (0, 0)),
            pl.BlockSpec((h2, H), lambda i: (0, 0)),
            pl.BlockSpec((1, H), lambda i: (0, 0)),
        ],
        out_specs=pl.BlockSpec((BR, H), lambda i: (i, 0)),
        out_shape=jax.ShapeDtypeStruct((N, H), jnp.float32),
    )(h, a0, a1, w1, b1.reshape(1, h2), bg.reshape(1, h2), bb.reshape(1, h2),
      bm.reshape(1, h2), bv.reshape(1, h2), w2, b2.reshape(1, H))


def _pool_body(x_ref, x1_ref, a0_ref, a1_ref, w1_ref, b1_ref, g_ref, bb_ref,
               m2_ref, v2_ref, w2_ref, b2_ref,
               wa_ref, wb_ref, wc_ref, sb_ref,
               batch_ref, attw_ref, c1w_ref, c1b_ref, cg_ref, cb_ref,
               cm_ref, cv_ref, c2w_ref, c2b_ref, c3w_ref, c3b_ref, o_ref):
    zg = x1_ref[...] + a0_ref[...] + a1_ref[...]
    tg = (jnp.dot(zg, w1_ref[...], preferred_element_type=jnp.float32)
          + b1_ref[...])
    scg = g_ref[...] * lax.rsqrt(v2_ref[...] + 1e-5)
    tg = jnp.maximum(tg * scg + (bb_ref[...] - m2_ref[...] * scg), 0.0)
    x2 = jnp.maximum(
        jnp.dot(tg, w2_ref[...], preferred_element_type=jnp.float32)
        + b2_ref[...], 0.0)
    xs = (
        jnp.dot(x_ref[...], wa_ref[...], preferred_element_type=jnp.float32)
        + jnp.dot(x1_ref[...], wb_ref[...], preferred_element_type=jnp.float32)
        + jnp.dot(x2, wc_ref[...], preferred_element_type=jnp.float32)
        + sb_ref[...])
    scores = lax.dot_general(attw_ref[...], xs, (((1,), (1,)), ((), ())),
                             preferred_element_type=jnp.float32)
    gid = lax.broadcasted_iota(jnp.int32, (G, N), 0)
    m = batch_ref[...] == gid
    sm = jnp.where(m, scores, -1e30)
    smax = jnp.max(sm, axis=1, keepdims=True)
    e = jnp.where(m, jnp.exp(sm - smax), 0.0)
    den = jnp.sum(e, axis=1, keepdims=True)
    aw = e / (den + 1e-16)
    x_att = jnp.dot(aw, xs, preferred_element_type=jnp.float32)
    mf = m.astype(jnp.float32)
    cnt = jnp.sum(mf, axis=1, keepdims=True)
    x_mean = (jnp.dot(mf, xs, preferred_element_type=jnp.float32)
              / jnp.maximum(cnt, 1.0))
    z = jnp.concatenate([x_att, x_mean], axis=1)
    sc1 = cg_ref[...] * lax.rsqrt(cv_ref[...] + 1e-5)
    z = jnp.maximum(
        (jnp.dot(z, c1w_ref[...], preferred_element_type=jnp.float32)
         + c1b_ref[...]) * sc1 + (cb_ref[...] - cm_ref[...] * sc1), 0.0)
    z = jnp.maximum(
        jnp.dot(z, c2w_ref[...], preferred_element_type=jnp.float32)
        + c2b_ref[...], 0.0)
    o_ref[...] = (jnp.dot(z, c3w_ref[...], preferred_element_type=jnp.float32)
                  + c3b_ref[...])


def _pool(x, x1, a0, a1, gw1, gb1, gg, gbb, gm, gv, gw2, gb2,
          wa, wb, wc, sb, batch2d, att_w, c1w, c1b, cg, cb, cm,
          cv, c2w, c2b, c3w, c3b):
    h2 = 2 * H
    full = lambda shape: pl.BlockSpec(shape, lambda: (0,) * len(shape))
    return pl.pallas_call(
        _pool_body,
        in_specs=[
            full((N, F_IN)), full((N, H)), full((N, H)), full((N, H)),
            full((H, h2)), full((1, h2)), full((1, h2)), full((1, h2)),
            full((1, h2)), full((1, h2)), full((h2, H)), full((1, H)),
            full((F_IN, H)), full((H, H)), full((H, H)), full((1, H)),
            full((1, N)), full((1, H)),
            full((h2, H)), full((1, H)),
            full((1, H)), full((1, H)), full((1, H)), full((1, H)),
            full((H, H // 2)), full((1, H // 2)),
            full((H // 2, 2)), full((1, 2)),
        ],
        out_specs=full((G, 2)),
        out_shape=jax.ShapeDtypeStruct((G, 2), jnp.float32),
    )(x, x1, a0, a1, gw1, gb1.reshape(1, h2), gg.reshape(1, h2),
      gbb.reshape(1, h2), gm.reshape(1, h2), gv.reshape(1, h2), gw2,
      gb2.reshape(1, H),
      wa, wb, wc, sb.reshape(1, H), batch2d, att_w, c1w,
      c1b.reshape(1, H), cg.reshape(1, H), cb.reshape(1, H),
      cm.reshape(1, H), cv.reshape(1, H), c2w,
      c2b.reshape(1, H // 2), c3w, c3b.reshape(1, 2))


def kernel(x, edge_index, batch, inp_W, inp_b, g1_l1_W, g1_l1_b, g1_bn_g,
           g1_bn_b, g1_bn_m, g1_bn_v, g1_l2_W, g1_l2_b, g2_l1_W, g2_l1_b,
           g2_bn_g, g2_bn_b, g2_bn_m, g2_bn_v, g2_l2_W, g2_l2_b, skip_W,
           skip_b, att_w, c1_W, c1_b, cbn_g, cbn_b, cbn_m, cbn_v, c2_W, c2_b,
           c3_W, c3_b):
    src_flat = edge_index[0]
    dst_flat = edge_index[1]

    h = _lin_relu(x, inp_W, inp_b)
    p1 = _sc_agg(h, src_flat, dst_flat)
    x1 = _gin(h, p1[0], p1[1], g1_l1_W, g1_l1_b, g1_bn_g, g1_bn_b, g1_bn_m,
              g1_bn_v, g1_l2_W, g1_l2_b)
    p2 = _sc_agg(x1, src_flat, dst_flat)
    return _pool(x, x1, p2[0], p2[1], g2_l1_W, g2_l1_b, g2_bn_g, g2_bn_b,
                 g2_bn_m, g2_bn_v, g2_l2_W, g2_l2_b,
                 skip_W[:F_IN], skip_W[F_IN:F_IN + H],
                 skip_W[F_IN + H:], skip_b, batch.reshape(1, N), att_w, c1_W,
                 c1_b, cbn_g, cbn_b, cbn_m, cbn_v, c2_W, c2_b, c3_W, c3_b)
